# baseline (speedup 1.0000x reference)
import jax
import jax.numpy as jnp
from jax.experimental import pallas as pl
from jax.experimental.pallas import tpu as pltpu

B, C, H, W = 8, 96, 128, 128
HW = H * W
NUM_CLASSES = 80
CP = 128
KK = 3 * CP
HID = 3 * C
OUT_CH = NUM_CLASSES + 4
MCH = 8192


def _fused_head_kernel(xb_ref, m_ref, w1_ref, b1_ref, w2c_ref, w2o_ref,
                       w2s_ref, b2_ref, cls_ref, off_ref, shp_ref,
                       xc_ref, acc_ref):
    @pl.when(pl.program_id(0) == 0)
    def _init():
        xc_ref[:, 0:W] = jnp.zeros((KK, W), jnp.bfloat16)
        xc_ref[:, W + HW:2 * W + HW] = jnp.zeros((KK, W), jnp.bfloat16)
        zstrip = jnp.zeros((CP - C, 2 * W + HW), jnp.bfloat16)
        for dx in range(3):
            xc_ref[dx * CP + C:(dx + 1) * CP, :] = zstrip
        xc_ref[0:C, W:W + 1] = jnp.zeros((C, 1), jnp.bfloat16)
        xc_ref[2 * CP:2 * CP + C, W + HW - 1:W + HW] = jnp.zeros((C, 1), jnp.bfloat16)

    xin = xb_ref[0].astype(jnp.bfloat16)
    xc_ref[CP:CP + C, W:W + HW] = xin
    xc_ref[0:C, W + 1:W + HW] = xin[:, 0:HW - 1] * m_ref[0:1, 1:HW]
    xc_ref[2 * CP:2 * CP + C, W:W + HW - 1] = xin[:, 1:HW] * m_ref[1:2, 0:HW - 1]

    b1 = b1_ref[:, 0:1]
    for m0 in range(0, HW, MCH):
        for dy in range(3):
            slab = xc_ref[:, dy * W + m0:dy * W + m0 + MCH]
            mm = jax.lax.dot_general(
                w1_ref[dy], slab, (((1,), (0,)), ((), ())),
                preferred_element_type=jnp.float32)
            if dy == 0:
                acc_ref[:, :] = mm
            else:
                acc_ref[:, :] = acc_ref[:, :] + mm
        hid = jnp.maximum(acc_ref[:, :] + b1, 0.0).astype(jnp.bfloat16)
        dn = (((1,), (0,)), ((), ()))
        cls_ref[0, :, m0:m0 + MCH] = jax.lax.dot_general(
            w2c_ref[:, :], hid[0:C], dn,
            preferred_element_type=jnp.float32) + b2_ref[0:NUM_CLASSES, 0:1]
        off_ref[0, :, m0:m0 + MCH] = jax.lax.dot_general(
            w2o_ref[:, :], hid[C:2 * C], dn,
            preferred_element_type=jnp.float32) + b2_ref[NUM_CLASSES:NUM_CLASSES + 2, 0:1]
        shp_ref[0, :, m0:m0 + MCH] = jax.lax.dot_general(
            w2s_ref[:, :], hid[2 * C:3 * C], dn,
            preferred_element_type=jnp.float32) + b2_ref[NUM_CLASSES + 2:, 0:1]


def _head_pallas(xb, m, w1_r, b1_col, w2c, w2o, w2s, b2_col):
    nb = xb.shape[0]
    cls, off, shp = pl.pallas_call(
        _fused_head_kernel,
        grid=(nb,),
        in_specs=[
            pl.BlockSpec((1, C, HW), lambda b: (b, 0, 0)),
            pl.BlockSpec((2, HW), lambda b: (0, 0)),
            pl.BlockSpec((3, HID, KK), lambda b: (0, 0, 0)),
            pl.BlockSpec((HID, 1), lambda b: (0, 0)),
            pl.BlockSpec((NUM_CLASSES, C), lambda b: (0, 0)),
            pl.BlockSpec((2, C), lambda b: (0, 0)),
            pl.BlockSpec((2, C), lambda b: (0, 0)),
            pl.BlockSpec((OUT_CH, 1), lambda b: (0, 0)),
        ],
        out_specs=[
            pl.BlockSpec((1, NUM_CLASSES, HW), lambda b: (b, 0, 0)),
            pl.BlockSpec((1, 2, HW), lambda b: (b, 0, 0)),
            pl.BlockSpec((1, 2, HW), lambda b: (b, 0, 0)),
        ],
        out_shape=[
            jax.ShapeDtypeStruct((nb, NUM_CLASSES, HW), jnp.float32),
            jax.ShapeDtypeStruct((nb, 2, HW), jnp.float32),
            jax.ShapeDtypeStruct((nb, 2, HW), jnp.float32),
        ],
        scratch_shapes=[
            pltpu.VMEM((KK, 2 * W + HW), jnp.bfloat16),
            pltpu.VMEM((HID, MCH), jnp.float32),
        ],
    )(xb, m, w1_r, b1_col, w2c, w2o, w2s, b2_col)
    cls = jnp.transpose(cls.reshape(nb, NUM_CLASSES, H, W), (0, 2, 3, 1))
    off = jnp.transpose(off.reshape(nb, 2, H, W), (0, 2, 3, 1))
    shp = jnp.transpose(shp.reshape(nb, 2, H, W), (0, 2, 3, 1))
    return cls, off, shp


def kernel(x, off_w1, off_b1, off_w2, off_b2, shp_w1, shp_b1, shp_w2, shp_b2,
           cls_w1, cls_b1, cls_w2, cls_b2):
    xb = x.reshape(B, C, HW)
    p = jnp.arange(HW, dtype=jnp.int32)
    m = jnp.stack([(p % W != 0), (p % W != W - 1)]).astype(jnp.bfloat16)
    w1_cat = jnp.concatenate([cls_w1, off_w1, shp_w1], axis=0)
    w1_k = jnp.transpose(w1_cat, (2, 3, 1, 0))
    w1_r = jnp.zeros((3, KK, HID), jnp.float32)
    for dx in range(3):
        w1_r = w1_r.at[:, dx * CP:dx * CP + C, :].set(w1_k[:, dx])
    w1_r = jnp.transpose(w1_r, (0, 2, 1)).astype(jnp.bfloat16)
    b1_col = jnp.concatenate([cls_b1, off_b1, shp_b1])[:, None]
    w2c = cls_w2[:, :, 0, 0].astype(jnp.bfloat16)
    w2o = off_w2[:, :, 0, 0].astype(jnp.bfloat16)
    w2s = shp_w2[:, :, 0, 0].astype(jnp.bfloat16)
    b2_col = jnp.concatenate([cls_b2, off_b2, shp_b2])[:, None]

    devs = jax.devices()
    nd = 1
    for d in (8, 4, 2):
        if len(devs) >= d and B % d == 0:
            nd = d
            break
    args = (xb, m, w1_r, b1_col, w2c, w2o, w2s, b2_col)
    if nd > 1:
        mesh = jax.sharding.Mesh(devs[:nd], ("b",))
        P = jax.sharding.PartitionSpec
        in_specs = (P("b"),) + (P(),) * 7
        out_specs = (P("b"), P("b"), P("b"))
        cls, off, shp = jax.shard_map(
            _head_pallas, mesh=mesh, in_specs=in_specs,
            out_specs=out_specs, check_vma=False)(*args)
    else:
        cls, off, shp = _head_pallas(*args)
    return cls, off, shp


# speedup vs baseline: 2.5478x; 2.5478x over previous
import jax
import jax.numpy as jnp
from jax.experimental import pallas as pl
from jax.experimental.pallas import tpu as pltpu

B, C, H, W = 8, 96, 128, 128
HW = H * W
NUM_CLASSES = 80
CP = 128
KK = 3 * CP
HID = 3 * C
OUT_CH = NUM_CLASSES + 4
MCH = 8192


def _fused_head_kernel(xb_ref, m_ref, w1_ref, b1_ref, w2c_ref, w2o_ref,
                       w2s_ref, b2_ref, cls_ref, off_ref, shp_ref,
                       xc_ref, acc_ref):
    @pl.when(pl.program_id(0) == 0)
    def _init():
        xc_ref[:, 0:W] = jnp.zeros((KK, W), jnp.bfloat16)
        xc_ref[:, W + HW:2 * W + HW] = jnp.zeros((KK, W), jnp.bfloat16)
        zstrip = jnp.zeros((CP - C, 2 * W + HW), jnp.bfloat16)
        for dx in range(3):
            xc_ref[dx * CP + C:(dx + 1) * CP, :] = zstrip
        xc_ref[0:C, W:W + 1] = jnp.zeros((C, 1), jnp.bfloat16)
        xc_ref[2 * CP:2 * CP + C, W + HW - 1:W + HW] = jnp.zeros((C, 1), jnp.bfloat16)

    xin = xb_ref[0].astype(jnp.bfloat16)
    xc_ref[CP:CP + C, W:W + HW] = xin
    xc_ref[0:C, W + 1:W + HW] = xin[:, 0:HW - 1] * m_ref[0:1, 1:HW]
    xc_ref[2 * CP:2 * CP + C, W:W + HW - 1] = xin[:, 1:HW] * m_ref[1:2, 0:HW - 1]

    b1 = b1_ref[:, 0:1]
    for m0 in range(0, HW, MCH):
        for dy in range(3):
            slab = xc_ref[:, dy * W + m0:dy * W + m0 + MCH]
            mm = jax.lax.dot_general(
                w1_ref[dy], slab, (((1,), (0,)), ((), ())),
                preferred_element_type=jnp.float32)
            if dy == 0:
                acc_ref[:, :] = mm
            else:
                acc_ref[:, :] = acc_ref[:, :] + mm
        hid = jnp.maximum(acc_ref[:, :] + b1, 0.0).astype(jnp.bfloat16)
        dn = (((1,), (0,)), ((), ()))
        out_c = jax.lax.dot_general(
            w2c_ref[:, :], hid[0:C], dn,
            preferred_element_type=jnp.float32) + b2_ref[0:NUM_CLASSES, 0:1]
        cls_ref[0, m0:m0 + MCH, :] = jnp.transpose(out_c)
        off_ref[0, :, m0:m0 + MCH] = jax.lax.dot_general(
            w2o_ref[:, :], hid[C:2 * C], dn,
            preferred_element_type=jnp.float32) + b2_ref[NUM_CLASSES:NUM_CLASSES + 2, 0:1]
        shp_ref[0, :, m0:m0 + MCH] = jax.lax.dot_general(
            w2s_ref[:, :], hid[2 * C:3 * C], dn,
            preferred_element_type=jnp.float32) + b2_ref[NUM_CLASSES + 2:, 0:1]


def kernel(x, off_w1, off_b1, off_w2, off_b2, shp_w1, shp_b1, shp_w2, shp_b2,
           cls_w1, cls_b1, cls_w2, cls_b2):
    xb = x.reshape(B, C, HW)
    p = jnp.arange(HW, dtype=jnp.int32)
    m = jnp.stack([(p % W != 0), (p % W != W - 1)]).astype(jnp.bfloat16)
    w1_cat = jnp.concatenate([cls_w1, off_w1, shp_w1], axis=0)
    w1_k = jnp.transpose(w1_cat, (2, 3, 1, 0))
    w1_r = jnp.zeros((3, KK, HID), jnp.float32)
    for dx in range(3):
        w1_r = w1_r.at[:, dx * CP:dx * CP + C, :].set(w1_k[:, dx])
    w1_r = jnp.transpose(w1_r, (0, 2, 1)).astype(jnp.bfloat16)
    b1_col = jnp.concatenate([cls_b1, off_b1, shp_b1])[:, None]
    w2c = cls_w2[:, :, 0, 0].astype(jnp.bfloat16)
    w2o = off_w2[:, :, 0, 0].astype(jnp.bfloat16)
    w2s = shp_w2[:, :, 0, 0].astype(jnp.bfloat16)
    b2_col = jnp.concatenate([cls_b2, off_b2, shp_b2])[:, None]

    cls, off, shp = pl.pallas_call(
        _fused_head_kernel,
        grid=(B,),
        in_specs=[
            pl.BlockSpec((1, C, HW), lambda b: (b, 0, 0)),
            pl.BlockSpec((2, HW), lambda b: (0, 0)),
            pl.BlockSpec((3, HID, KK), lambda b: (0, 0, 0)),
            pl.BlockSpec((HID, 1), lambda b: (0, 0)),
            pl.BlockSpec((NUM_CLASSES, C), lambda b: (0, 0)),
            pl.BlockSpec((2, C), lambda b: (0, 0)),
            pl.BlockSpec((2, C), lambda b: (0, 0)),
            pl.BlockSpec((OUT_CH, 1), lambda b: (0, 0)),
        ],
        out_specs=[
            pl.BlockSpec((1, HW, NUM_CLASSES), lambda b: (b, 0, 0)),
            pl.BlockSpec((1, 2, HW), lambda b: (b, 0, 0)),
            pl.BlockSpec((1, 2, HW), lambda b: (b, 0, 0)),
        ],
        out_shape=[
            jax.ShapeDtypeStruct((B, HW, NUM_CLASSES), jnp.float32),
            jax.ShapeDtypeStruct((B, 2, HW), jnp.float32),
            jax.ShapeDtypeStruct((B, 2, HW), jnp.float32),
        ],
        scratch_shapes=[
            pltpu.VMEM((KK, 2 * W + HW), jnp.bfloat16),
            pltpu.VMEM((HID, MCH), jnp.float32),
        ],
    )(xb, m, w1_r, b1_col, w2c, w2o, w2s, b2_col)
    cls = cls.reshape(B, H, W, NUM_CLASSES)
    off = jnp.transpose(off.reshape(B, 2, H, W), (0, 2, 3, 1))
    shp = jnp.transpose(shp.reshape(B, 2, H, W), (0, 2, 3, 1))
    return cls, off, shp


# speedup vs baseline: 2.9307x; 1.1503x over previous
import jax
import jax.numpy as jnp
from jax.experimental import pallas as pl
from jax.experimental.pallas import tpu as pltpu

B, C, H, W = 8, 96, 128, 128
HW = H * W
NUM_CLASSES = 80
CP = 128
KK = 3 * CP
HID = 3 * C
OUT_CH = NUM_CLASSES + 4
MCH = 8192


def _fused_head_kernel(xb_ref, m_ref, w1_ref, b1_ref, w2c_ref, w2o_ref,
                       w2s_ref, b2_ref, cls_ref, off_ref, shp_ref, xc_ref):
    @pl.when(pl.program_id(0) == 0)
    def _init():
        xc_ref[:, 0:W] = jnp.zeros((KK, W), jnp.bfloat16)
        xc_ref[:, W + HW:2 * W + HW] = jnp.zeros((KK, W), jnp.bfloat16)
        zstrip = jnp.zeros((CP - C, 2 * W + HW), jnp.bfloat16)
        for dx in range(3):
            xc_ref[dx * CP + C:(dx + 1) * CP, :] = zstrip
        xc_ref[0:C, W:W + 1] = jnp.zeros((C, 1), jnp.bfloat16)
        xc_ref[2 * CP:2 * CP + C, W + HW - 1:W + HW] = jnp.zeros((C, 1), jnp.bfloat16)

    xin = xb_ref[0].astype(jnp.bfloat16)
    xc_ref[CP:CP + C, W:W + HW] = xin
    xc_ref[0:C, W + 1:W + HW] = xin[:, 0:HW - 1] * m_ref[0:1, 1:HW]
    xc_ref[2 * CP:2 * CP + C, W:W + HW - 1] = xin[:, 1:HW] * m_ref[1:2, 0:HW - 1]

    b1 = b1_ref[:, 0:1]
    for m0 in range(0, HW, MCH):
        acc = None
        for dy in range(3):
            slab = xc_ref[:, dy * W + m0:dy * W + m0 + MCH]
            mm = jax.lax.dot_general(
                w1_ref[dy], slab, (((1,), (0,)), ((), ())),
                preferred_element_type=jnp.float32)
            acc = mm if acc is None else acc + mm
        hid = jnp.maximum(acc + b1, 0.0).astype(jnp.bfloat16)
        dn = (((1,), (0,)), ((), ()))
        cls_ref[0, :, m0:m0 + MCH] = jax.lax.dot_general(
            w2c_ref[:, :], hid[0:C], dn,
            preferred_element_type=jnp.float32) + b2_ref[0:NUM_CLASSES, 0:1]
        off_ref[0, :, m0:m0 + MCH] = jax.lax.dot_general(
            w2o_ref[:, :], hid[C:2 * C], dn,
            preferred_element_type=jnp.float32) + b2_ref[NUM_CLASSES:NUM_CLASSES + 2, 0:1]
        shp_ref[0, :, m0:m0 + MCH] = jax.lax.dot_general(
            w2s_ref[:, :], hid[2 * C:3 * C], dn,
            preferred_element_type=jnp.float32) + b2_ref[NUM_CLASSES + 2:, 0:1]


def kernel(x, off_w1, off_b1, off_w2, off_b2, shp_w1, shp_b1, shp_w2, shp_b2,
           cls_w1, cls_b1, cls_w2, cls_b2):
    xb = x.reshape(B, C, HW)
    p = jnp.arange(HW, dtype=jnp.int32)
    m = jnp.stack([(p % W != 0), (p % W != W - 1)]).astype(jnp.bfloat16)
    w1_cat = jnp.concatenate([cls_w1, off_w1, shp_w1], axis=0)
    w1_k = jnp.transpose(w1_cat, (2, 3, 1, 0))
    w1_r = jnp.zeros((3, KK, HID), jnp.float32)
    for dx in range(3):
        w1_r = w1_r.at[:, dx * CP:dx * CP + C, :].set(w1_k[:, dx])
    w1_r = jnp.transpose(w1_r, (0, 2, 1)).astype(jnp.bfloat16)
    b1_col = jnp.concatenate([cls_b1, off_b1, shp_b1])[:, None]
    w2c = cls_w2[:, :, 0, 0].astype(jnp.bfloat16)
    w2o = off_w2[:, :, 0, 0].astype(jnp.bfloat16)
    w2s = shp_w2[:, :, 0, 0].astype(jnp.bfloat16)
    b2_col = jnp.concatenate([cls_b2, off_b2, shp_b2])[:, None]

    cls, off, shp = pl.pallas_call(
        _fused_head_kernel,
        grid=(B,),
        in_specs=[
            pl.BlockSpec((1, C, HW), lambda b: (b, 0, 0)),
            pl.BlockSpec((2, HW), lambda b: (0, 0)),
            pl.BlockSpec((3, HID, KK), lambda b: (0, 0, 0)),
            pl.BlockSpec((HID, 1), lambda b: (0, 0)),
            pl.BlockSpec((NUM_CLASSES, C), lambda b: (0, 0)),
            pl.BlockSpec((2, C), lambda b: (0, 0)),
            pl.BlockSpec((2, C), lambda b: (0, 0)),
            pl.BlockSpec((OUT_CH, 1), lambda b: (0, 0)),
        ],
        out_specs=[
            pl.BlockSpec((1, NUM_CLASSES, HW), lambda b: (b, 0, 0)),
            pl.BlockSpec((1, 2, HW), lambda b: (b, 0, 0)),
            pl.BlockSpec((1, 2, HW), lambda b: (b, 0, 0)),
        ],
        out_shape=[
            jax.ShapeDtypeStruct((B, NUM_CLASSES, HW), jnp.float32),
            jax.ShapeDtypeStruct((B, 2, HW), jnp.float32),
            jax.ShapeDtypeStruct((B, 2, HW), jnp.float32),
        ],
        scratch_shapes=[
            pltpu.VMEM((KK, 2 * W + HW), jnp.bfloat16),
        ],
    )(xb, m, w1_r, b1_col, w2c, w2o, w2s, b2_col)
    cls = jnp.transpose(cls.reshape(B, NUM_CLASSES, H, W), (0, 2, 3, 1))
    off = jnp.transpose(off.reshape(B, 2, H, W), (0, 2, 3, 1))
    shp = jnp.transpose(shp.reshape(B, 2, H, W), (0, 2, 3, 1))
    return cls, off, shp


# speedup vs baseline: 2.9320x; 1.0004x over previous
import jax
import jax.numpy as jnp
from jax.experimental import pallas as pl
from jax.experimental.pallas import tpu as pltpu

B, C, H, W = 8, 96, 128, 128
HW = H * W
NUM_CLASSES = 80
CP = 96
KK = 3 * CP
HID = 3 * C
OUT_CH = NUM_CLASSES + 4
MCH = 8192


def _fused_head_kernel(xb_ref, m_ref, w1_ref, b1_ref, w2c_ref, w2o_ref,
                       w2s_ref, b2_ref, cls_ref, off_ref, shp_ref, xc_ref):
    @pl.when(pl.program_id(0) == 0)
    def _init():
        xc_ref[:, 0:W] = jnp.zeros((KK, W), jnp.bfloat16)
        xc_ref[:, W + HW:2 * W + HW] = jnp.zeros((KK, W), jnp.bfloat16)
        xc_ref[0:C, W:W + 1] = jnp.zeros((C, 1), jnp.bfloat16)
        xc_ref[2 * CP:2 * CP + C, W + HW - 1:W + HW] = jnp.zeros((C, 1), jnp.bfloat16)

    xin = xb_ref[0].astype(jnp.bfloat16)
    xc_ref[CP:CP + C, W:W + HW] = xin
    xc_ref[0:C, W + 1:W + HW] = xin[:, 0:HW - 1] * m_ref[0:1, 1:HW]
    xc_ref[2 * CP:2 * CP + C, W:W + HW - 1] = xin[:, 1:HW] * m_ref[1:2, 0:HW - 1]

    b1 = b1_ref[:, 0:1]
    for m0 in range(0, HW, MCH):
        acc = None
        for dy in range(3):
            slab = xc_ref[:, dy * W + m0:dy * W + m0 + MCH]
            mm = jax.lax.dot_general(
                w1_ref[dy], slab, (((1,), (0,)), ((), ())),
                preferred_element_type=jnp.float32)
            acc = mm if acc is None else acc + mm
        hid = jnp.maximum(acc + b1, 0.0).astype(jnp.bfloat16)
        dn = (((1,), (0,)), ((), ()))
        cls_ref[0, :, m0:m0 + MCH] = jax.lax.dot_general(
            w2c_ref[:, :], hid[0:C], dn,
            preferred_element_type=jnp.float32) + b2_ref[0:NUM_CLASSES, 0:1]
        off_ref[0, :, m0:m0 + MCH] = jax.lax.dot_general(
            w2o_ref[:, :], hid[C:2 * C], dn,
            preferred_element_type=jnp.float32) + b2_ref[NUM_CLASSES:NUM_CLASSES + 2, 0:1]
        shp_ref[0, :, m0:m0 + MCH] = jax.lax.dot_general(
            w2s_ref[:, :], hid[2 * C:3 * C], dn,
            preferred_element_type=jnp.float32) + b2_ref[NUM_CLASSES + 2:, 0:1]


def kernel(x, off_w1, off_b1, off_w2, off_b2, shp_w1, shp_b1, shp_w2, shp_b2,
           cls_w1, cls_b1, cls_w2, cls_b2):
    xb = x.reshape(B, C, HW)
    p = jnp.arange(HW, dtype=jnp.int32)
    m = jnp.stack([(p % W != 0), (p % W != W - 1)]).astype(jnp.bfloat16)
    w1_cat = jnp.concatenate([cls_w1, off_w1, shp_w1], axis=0)
    w1_k = jnp.transpose(w1_cat, (2, 3, 1, 0))
    w1_r = w1_k.reshape(3, KK, HID)
    w1_r = jnp.transpose(w1_r, (0, 2, 1)).astype(jnp.bfloat16)
    b1_col = jnp.concatenate([cls_b1, off_b1, shp_b1])[:, None]
    w2c = cls_w2[:, :, 0, 0].astype(jnp.bfloat16)
    w2o = off_w2[:, :, 0, 0].astype(jnp.bfloat16)
    w2s = shp_w2[:, :, 0, 0].astype(jnp.bfloat16)
    b2_col = jnp.concatenate([cls_b2, off_b2, shp_b2])[:, None]

    cls, off, shp = pl.pallas_call(
        _fused_head_kernel,
        grid=(B,),
        in_specs=[
            pl.BlockSpec((1, C, HW), lambda b: (b, 0, 0)),
            pl.BlockSpec((2, HW), lambda b: (0, 0)),
            pl.BlockSpec((3, HID, KK), lambda b: (0, 0, 0)),
            pl.BlockSpec((HID, 1), lambda b: (0, 0)),
            pl.BlockSpec((NUM_CLASSES, C), lambda b: (0, 0)),
            pl.BlockSpec((2, C), lambda b: (0, 0)),
            pl.BlockSpec((2, C), lambda b: (0, 0)),
            pl.BlockSpec((OUT_CH, 1), lambda b: (0, 0)),
        ],
        out_specs=[
            pl.BlockSpec((1, NUM_CLASSES, HW), lambda b: (b, 0, 0)),
            pl.BlockSpec((1, 2, HW), lambda b: (b, 0, 0)),
            pl.BlockSpec((1, 2, HW), lambda b: (b, 0, 0)),
        ],
        out_shape=[
            jax.ShapeDtypeStruct((B, NUM_CLASSES, HW), jnp.float32),
            jax.ShapeDtypeStruct((B, 2, HW), jnp.float32),
            jax.ShapeDtypeStruct((B, 2, HW), jnp.float32),
        ],
        scratch_shapes=[
            pltpu.VMEM((KK, 2 * W + HW), jnp.bfloat16),
        ],
    )(xb, m, w1_r, b1_col, w2c, w2o, w2s, b2_col)
    cls = jnp.transpose(cls.reshape(B, NUM_CLASSES, H, W), (0, 2, 3, 1))
    off = jnp.transpose(off.reshape(B, 2, H, W), (0, 2, 3, 1))
    shp = jnp.transpose(shp.reshape(B, 2, H, W), (0, 2, 3, 1))
    return cls, off, shp
